# baseline (speedup 1.0000x reference)
import jax
import jax.numpy as jnp
from jax import lax
from jax.experimental import pallas as pl
from jax.experimental.pallas import tpu as pltpu
from jax.experimental.pallas import tpu_sc as plsc

FP = 16
LANES = 128
KDEPTH = 5
NSC = 2
NTILE = 16
BLK = 2000


def _sc_mesh():
    return plsc.VectorSubcoreMesh(
        core_axis_name="c", subcore_axis_name="s", num_cores=NSC,
        num_subcores=NTILE)


def _agg_loop(tab, edg, idxb, rowsb, acc, gsem, ssem, isem, slab_lo, nslab):
    def idx_load(t):
        p = lax.rem(t, 3)
        r0 = (slab_lo + t) * KDEPTH
        pltpu.async_copy(edg.at[0, pl.ds(r0, KDEPTH)], idxb.at[p, 0], isem)
        pltpu.async_copy(edg.at[1, pl.ds(r0, KDEPTH)], idxb.at[p, 1], isem)

    def gathers(t):
        p = lax.rem(t, 3)
        b = lax.rem(t, 2)
        for j in range(KDEPTH):
            pltpu.async_copy(tab.at[idxb.at[p, 0, j]], rowsb.at[b, j], gsem)

    def drain_scatters():
        for j in range(KDEPTH):
            pltpu.make_async_copy(rowsb.at[0, j], acc.at[idxb.at[0, 1, j]],
                                  ssem).wait()

    r_start = slab_lo * KDEPTH
    pltpu.sync_copy(edg.at[0, pl.ds(r_start, KDEPTH)], idxb.at[0, 0])
    pltpu.sync_copy(edg.at[1, pl.ds(r_start, KDEPTH)], idxb.at[0, 1])
    gathers(0)
    idx_load(1)

    def slab(t, carry):
        p = lax.rem(t, 3)
        b = lax.rem(t, 2)
        for j in range(KDEPTH):
            pltpu.make_async_copy(tab.at[idxb.at[p, 0, j]], rowsb.at[b, j],
                                  gsem).wait()
        for j in range(KDEPTH):
            pltpu.async_copy(rowsb.at[b, j], acc.at[idxb.at[p, 1, j]], ssem,
                             add=True)

        @pl.when(t + 1 < nslab)
        def _():
            pltpu.make_async_copy(edg.at[0, pl.ds(r_start, KDEPTH)],
                                  idxb.at[0, 0], isem).wait()
            pltpu.make_async_copy(edg.at[1, pl.ds(r_start, KDEPTH)],
                                  idxb.at[0, 1], isem).wait()

        @pl.when(t >= 1)
        def _():
            drain_scatters()

        @pl.when(t + 2 < nslab)
        def _():
            idx_load(t + 2)

        @pl.when(t + 1 < nslab)
        def _():
            gathers(t + 1)

        return carry

    lax.fori_loop(0, nslab, slab, 0)
    drain_scatters()


def _sc_scratch(np_):
    return [
        pltpu.VMEM((3, 2, KDEPTH, LANES), jnp.int32),
        pltpu.VMEM((2, KDEPTH, LANES, FP), jnp.float32),
        pltpu.VMEM_SHARED((np_, FP), jnp.float32),
        pltpu.SemaphoreType.DMA,
        pltpu.SemaphoreType.DMA,
        pltpu.SemaphoreType.DMA,
    ]


def _make_sc_agg1(np_, rows):
    stripe = np_ // NTILE
    slabs_per_sc = rows // (NSC * KDEPTH)

    def body(tab, edg, zrows, out_a, out_b, idxb, rowsb, acc, gsem,
             ssem, isem):
        c = lax.axis_index("c")
        s = lax.axis_index("s")
        pltpu.sync_copy(zrows, acc.at[pl.ds(s * stripe, stripe)])
        plsc.subcore_barrier()
        lo = c * slabs_per_sc + (s * slabs_per_sc) // NTILE
        hi = c * slabs_per_sc + ((s + 1) * slabs_per_sc) // NTILE
        _agg_loop(tab, edg, idxb, rowsb, acc, gsem, ssem, isem, lo, hi - lo)
        plsc.subcore_barrier()

        @pl.when(c == 0)
        def _():
            pltpu.sync_copy(acc.at[pl.ds(s * stripe, stripe)],
                            out_a.at[pl.ds(s * stripe, stripe)])

        @pl.when(c == 1)
        def _():
            pltpu.sync_copy(acc.at[pl.ds(s * stripe, stripe)],
                            out_b.at[pl.ds(s * stripe, stripe)])

    return pl.kernel(
        body,
        out_type=(jax.ShapeDtypeStruct((np_, FP), jnp.float32),
                  jax.ShapeDtypeStruct((np_, FP), jnp.float32)),
        mesh=_sc_mesh(),
        compiler_params=pltpu.CompilerParams(use_tc_tiling_on_sc=False),
        scratch_types=_sc_scratch(np_),
    )


def _make_sc_agg2(np_, rows):
    stripe = np_ // NTILE
    nslabs = rows // KDEPTH

    def body(tab_a, tab_b, edg, zrows, out_a, out_b, idxb, rowsb, acc,
             gsem, ssem, isem):
        c = lax.axis_index("c")
        s = lax.axis_index("s")
        pltpu.sync_copy(zrows, acc.at[pl.ds(s * stripe, stripe)])
        plsc.subcore_barrier()
        lo = (s * nslabs) // NTILE
        hi = ((s + 1) * nslabs) // NTILE

        @pl.when(c == 0)
        def _():
            _agg_loop(tab_a, edg, idxb, rowsb, acc, gsem, ssem, isem,
                      lo, hi - lo)

        @pl.when(c == 1)
        def _():
            _agg_loop(tab_b, edg, idxb, rowsb, acc, gsem, ssem, isem,
                      lo, hi - lo)

        plsc.subcore_barrier()

        @pl.when(c == 0)
        def _():
            pltpu.sync_copy(acc.at[pl.ds(s * stripe, stripe)],
                            out_a.at[pl.ds(s * stripe, stripe)])

        @pl.when(c == 1)
        def _():
            pltpu.sync_copy(acc.at[pl.ds(s * stripe, stripe)],
                            out_b.at[pl.ds(s * stripe, stripe)])

    return pl.kernel(
        body,
        out_type=(jax.ShapeDtypeStruct((np_, FP), jnp.float32),
                  jax.ShapeDtypeStruct((np_, FP), jnp.float32)),
        mesh=_sc_mesh(),
        compiler_params=pltpu.CompilerParams(use_tc_tiling_on_sc=False),
        scratch_types=_sc_scratch(np_),
    )


def _tc1_body(n, a0, a1, xp, wl, wr, b, g, be, ha, hb, inv_ref, ssum, ssq):
    ph = pl.program_id(0)
    i = pl.program_id(1)
    cnt = a0[:, FP - 1:FP] + a1[:, FP - 1:FP]
    inv = 1.0 / jnp.maximum(cnt, 1.0)
    agg = (a0[...] + a1[...]) * inv
    t = (jnp.dot(agg, wl[...], preferred_element_type=jnp.float32)
         + jnp.dot(xp[...], wr[...], preferred_element_type=jnp.float32)
         + b[...])

    @pl.when((ph == 0) & (i == 0))
    def _():
        ssum[...] = jnp.zeros_like(ssum)
        ssq[...] = jnp.zeros_like(ssq)

    @pl.when(ph == 0)
    def _():
        ssum[...] += jnp.sum(t, axis=0, keepdims=True)
        ssq[...] += jnp.sum(t * t, axis=0, keepdims=True)

    rn = 1.0 / n
    mean = ssum[...] * rn
    var = ssq[...] * rn - mean * mean
    isd = lax.rsqrt(var + 1e-5)
    h = jnp.maximum((t - mean) * isd * g[...] + be[...], 0.0)
    ha[...] = h[:, :FP]
    hb[...] = h[:, FP:]
    inv_ref[...] = inv


def _tc2_body(n, a2a, a2b, h1a, h1b, invc, wl, wr, b, g, be, wm1, bm1, wm2,
              bm2, out, ssum, ssq):
    ph = pl.program_id(0)
    i = pl.program_id(1)
    inv = invc[...]
    t = (jnp.dot(a2a[...] * inv, wl[0:FP, :],
                 preferred_element_type=jnp.float32)
         + jnp.dot(a2b[...] * inv, wl[FP:2 * FP, :],
                   preferred_element_type=jnp.float32)
         + jnp.dot(h1a[...], wr[0:FP, :],
                   preferred_element_type=jnp.float32)
         + jnp.dot(h1b[...], wr[FP:2 * FP, :],
                   preferred_element_type=jnp.float32)
         + b[...])

    @pl.when((ph == 0) & (i == 0))
    def _():
        ssum[...] = jnp.zeros_like(ssum)
        ssq[...] = jnp.zeros_like(ssq)

    @pl.when(ph == 0)
    def _():
        ssum[...] += jnp.sum(t, axis=0, keepdims=True)
        ssq[...] += jnp.sum(t * t, axis=0, keepdims=True)

    rn = 1.0 / n
    mean = ssum[...] * rn
    var = ssq[...] * rn - mean * mean
    isd = lax.rsqrt(var + 1e-5)
    h = jnp.maximum((t - mean) * isd * g[...] + be[...], 0.0)
    m = jnp.maximum(
        jnp.dot(h, wm1[...], preferred_element_type=jnp.float32) + bm1[...],
        0.0)
    out[...] = (jnp.dot(m, wm2[...], preferred_element_type=jnp.float32)
                + bm2[...])


def kernel(x, edge_index, W1l, b1l, W1r, g1, be1, W2l, b2l, W2r, g2, be2,
           Wm1, bm1, Wm2, bm2):
    import functools
    n, f = x.shape
    e = edge_index.shape[1]
    nb = n // BLK
    np_ = (n // 128 + 1) * 128
    rows = e // LANES

    edg = edge_index.astype(jnp.int32).reshape(2, rows, LANES)

    xp = jnp.concatenate(
        [x, jnp.zeros((n, FP - 1 - f), x.dtype), jnp.ones((n, 1), x.dtype)],
        axis=1)
    zrows = jnp.zeros((np_ // NTILE, FP), jnp.float32)
    w1l = jnp.concatenate([W1l, jnp.zeros((FP - f, W1l.shape[1]))], axis=0)
    w1r = jnp.concatenate([W1r, jnp.zeros((FP - f, W1r.shape[1]))], axis=0)

    a1a, a1b = _make_sc_agg1(np_, rows)(xp, edg, zrows)

    row_spec = pl.BlockSpec((BLK, FP), lambda ph, i: (i, 0))
    col_spec = pl.BlockSpec((BLK, 1), lambda ph, i: (i, 0))
    full = lambda shape: pl.BlockSpec(shape, lambda ph, i: (0, 0))

    h1a, h1b, invc = pl.pallas_call(
        functools.partial(_tc1_body, n),
        grid=(2, nb),
        in_specs=[row_spec, row_spec, row_spec, full((FP, 32)),
                  full((FP, 32)), full((1, 32)), full((1, 32)),
                  full((1, 32))],
        out_specs=[row_spec, row_spec, col_spec],
        out_shape=[jax.ShapeDtypeStruct((n, FP), jnp.float32),
                   jax.ShapeDtypeStruct((n, FP), jnp.float32),
                   jax.ShapeDtypeStruct((n, 1), jnp.float32)],
        scratch_shapes=[pltpu.VMEM((1, 32), jnp.float32),
                        pltpu.VMEM((1, 32), jnp.float32)],
    )(a1a, a1b, xp, w1l, w1r, b1l.reshape(1, -1), g1.reshape(1, -1),
      be1.reshape(1, -1))

    a2a, a2b = _make_sc_agg2(np_, rows)(h1a, h1b, edg, zrows)

    out = pl.pallas_call(
        functools.partial(_tc2_body, n),
        grid=(2, nb),
        in_specs=[row_spec, row_spec, row_spec, row_spec, col_spec,
                  full((32, 32)), full((32, 32)), full((1, 32)),
                  full((1, 32)), full((1, 32)), full((32, 64)),
                  full((1, 64)), full((64, 1)), full((1, 1))],
        out_specs=col_spec,
        out_shape=jax.ShapeDtypeStruct((n, 1), jnp.float32),
        scratch_shapes=[pltpu.VMEM((1, 32), jnp.float32),
                        pltpu.VMEM((1, 32), jnp.float32)],
    )(a2a, a2b, h1a, h1b, invc, W2l, W2r, b2l.reshape(1, -1),
      g2.reshape(1, -1), be2.reshape(1, -1), Wm1, bm1.reshape(1, -1), Wm2,
      bm2.reshape(1, -1))

    return out.reshape(n)
